# baseline (speedup 1.0000x reference)
import functools

import jax
import jax.numpy as jnp
from jax import lax
from jax.experimental import pallas as pl
from jax.experimental.pallas import tpu as pltpu
from jax.experimental.pallas import tpu_sc as plsc

N = 10000
E = 320000
NFEAT = 128
HIDDEN = 128
NCLASS = 16

NC = 2
NS = 16
NW = NC * NS
CHUNK = 128

NP = 10240
ROWS_PER_TILE = NP // NS
EPT = 10240
E_PAD = EPT * NW
NCH = EPT // CHUNK
DSEM = 8

ECHUNK = 64
ENCH = EPT // ECHUNK
NBUF = 4
LOOK = 2
NG = ENCH // NBUF

_mesh = plsc.VectorSubcoreMesh(core_axis_name="c", subcore_axis_name="s")


def _fill_2d(ref, rows, value):
    vec = jnp.full((16,), value, dtype=jnp.float32)

    def body(i, _):
        r = i // 8
        col = (i % 8) * 16
        ref[r, pl.ds(col, 16)] = vec
        return 0

    lax.fori_loop(0, rows * 8, body, 0)


@functools.partial(
    pl.kernel,
    mesh=_mesh,
    out_type=jax.ShapeDtypeStruct((NC * NP,), jnp.float32),
    scratch_types=[
        pltpu.VMEM((CHUNK,), jnp.float32),
        pltpu.VMEM((NCH, CHUNK), jnp.int32),
        pltpu.VMEM((ROWS_PER_TILE,), jnp.float32),
        pltpu.VMEM_SHARED((NP,), jnp.float32),
    ] + [pltpu.SemaphoreType.DMA] * DSEM,
)
def _deg_kernel(dst_hbm, out_hbm, ones_v, idx_v, zero_v, acc_sh, *sems):
    c = lax.axis_index("c")
    s = lax.axis_index("s")
    wid = s * NC + c
    one = jnp.full((16,), 1.0, dtype=jnp.float32)
    zero = jnp.zeros((16,), dtype=jnp.float32)

    def fill_ones(i, _):
        ones_v[pl.ds(i * 16, 16)] = one
        return 0

    lax.fori_loop(0, CHUNK // 16, fill_ones, 0)

    def fill_zero(i, _):
        zero_v[pl.ds(i * 16, 16)] = zero
        return 0

    lax.fori_loop(0, ROWS_PER_TILE // 16, fill_zero, 0)
    pltpu.sync_copy(zero_v, acc_sh.at[pl.ds(s * ROWS_PER_TILE, ROWS_PER_TILE)])
    pltpu.sync_copy(dst_hbm.at[wid], idx_v)
    plsc.subcore_barrier()

    def grp_body(grp, _):
        for b in range(DSEM):
            i = grp * DSEM + b

            @pl.when(i >= DSEM)
            def _():
                pltpu.make_async_copy(ones_v, acc_sh.at[idx_v.at[0]],
                                      sems[b]).wait()

            pltpu.async_copy(ones_v, acc_sh.at[idx_v.at[i]], sems[b],
                             add=True)
        return 0

    lax.fori_loop(0, NCH // DSEM, grp_body, 0)
    for b in range(DSEM):
        pltpu.make_async_copy(ones_v, acc_sh.at[idx_v.at[0]], sems[b]).wait()
    plsc.subcore_barrier()

    row0 = s * ROWS_PER_TILE
    pltpu.sync_copy(
        acc_sh.at[pl.ds(row0, ROWS_PER_TILE)],
        out_hbm.at[pl.ds(c * NP + row0, ROWS_PER_TILE)],
    )


@functools.partial(
    pl.kernel,
    mesh=_mesh,
    out_type=jax.ShapeDtypeStruct((NC * NP, HIDDEN), jnp.float32),
    scratch_types=[pltpu.VMEM((ECHUNK, HIDDEN), jnp.float32)] * NBUF
      + [pltpu.VMEM((ECHUNK,), jnp.int32)] * NBUF
      + [pltpu.VMEM((ECHUNK,), jnp.int32)] * NBUF
      + [pltpu.VMEM((ECHUNK,), jnp.int32)] * NBUF
      + [pltpu.VMEM_SHARED((NP, HIDDEN), jnp.float32)]
      + [pltpu.SemaphoreType.DMA] * (3 * NBUF),
)
def _edge_kernel(g_hbm, pk_hbm, out_hbm, *rest):
    rows = list(rest[:NBUF])
    pk_r = list(rest[NBUF:2 * NBUF])
    idxs_r = list(rest[2 * NBUF:3 * NBUF])
    idxd_r = list(rest[3 * NBUF:4 * NBUF])
    acc_sh = rest[4 * NBUF]
    sems = rest[4 * NBUF + 1:]
    gsem = sems[:NBUF]
    ssem = sems[NBUF:2 * NBUF]
    isem = sems[2 * NBUF:]
    c = lax.axis_index("c")
    s = lax.axis_index("s")
    wid = s * NC + c

    _fill_2d(rows[0], ECHUNK, 0.0)
    row0 = s * ROWS_PER_TILE

    def zbody(i, _):
        pltpu.sync_copy(rows[0],
                        acc_sh.at[pl.ds(row0 + i * ECHUNK, ECHUNK), :])
        return 0

    lax.fori_loop(0, ROWS_PER_TILE // ECHUNK, zbody, 0)

    for b in range(NBUF):
        pltpu.async_copy(pk_hbm.at[wid, b], pk_r[b], isem[b])
    plsc.subcore_barrier()

    def unpack(j, b):
        pltpu.make_async_copy(pk_hbm.at[wid, 0], pk_r[b], isem[b]).wait()

        def ub(k, _):
            v = pk_r[b][pl.ds(k * 16, 16)]
            idxs_r[b][pl.ds(k * 16, 16)] = lax.bitwise_and(v, 0x3FFF)
            idxd_r[b][pl.ds(k * 16, 16)] = lax.shift_right_logical(v, 14)
            return 0

        lax.fori_loop(0, ECHUNK // 16, ub, 0)

        @pl.when(j + NBUF < ENCH)
        def _():
            pltpu.async_copy(pk_hbm.at[wid, j + NBUF], pk_r[b], isem[b])

    def scatter_wait(b):
        pltpu.make_async_copy(rows[b], acc_sh.at[idxd_r[b]], ssem[b]).wait()

    def gather_wait(b):
        pltpu.make_async_copy(g_hbm.at[idxs_r[b]], rows[b], gsem[b]).wait()

    for b in range(LOOK):
        unpack(b, b)
        pltpu.async_copy(g_hbm.at[idxs_r[b]], rows[b], gsem[b])

    def grp_body(grp, _):
        for b in range(NBUF):
            i = grp * NBUF + b
            bl = (b + LOOK) % NBUF

            @pl.when(i + LOOK < ENCH)
            def _():
                @pl.when(i + LOOK >= NBUF)
                def _():
                    scatter_wait(bl)

                unpack(i + LOOK, bl)
                pltpu.async_copy(g_hbm.at[idxs_r[bl]], rows[bl], gsem[bl])

            gather_wait(b)
            pltpu.async_copy(rows[b], acc_sh.at[idxd_r[b]], ssem[b],
                             add=True)
        return 0

    lax.fori_loop(0, NG, grp_body, 0)
    for b in range(NBUF):
        scatter_wait(b)
    plsc.subcore_barrier()

    def obody(i, _):
        r = row0 + i * ECHUNK
        pltpu.sync_copy(acc_sh.at[pl.ds(r, ECHUNK), :],
                        out_hbm.at[pl.ds(c * NP + r, ECHUNK), :])
        return 0

    lax.fori_loop(0, ROWS_PER_TILE // ECHUNK, obody, 0)


BR = 640


def _dense1_body(cnt_ref, x_ref, w1_ref, g_ref, dinv_ref):
    deg = cnt_ref[0] + cnt_ref[1] + 1.0
    dinv = lax.rsqrt(deg)
    h = jnp.dot(x_ref[...], w1_ref[...], preferred_element_type=jnp.float32)
    g_ref[...] = h * dinv
    dinv_ref[...] = dinv


def _dense1(cnt, x_pad, W1):
    return pl.pallas_call(
        _dense1_body,
        grid=(NP // BR,),
        in_specs=[
            pl.BlockSpec((2, BR, 1), lambda i: (0, i, 0)),
            pl.BlockSpec((BR, NFEAT), lambda i: (i, 0)),
            pl.BlockSpec((NFEAT, HIDDEN), lambda i: (0, 0)),
        ],
        out_specs=[
            pl.BlockSpec((BR, HIDDEN), lambda i: (i, 0)),
            pl.BlockSpec((BR, 1), lambda i: (i, 0)),
        ],
        out_shape=[
            jax.ShapeDtypeStruct((NP, HIDDEN), jnp.float32),
            jax.ShapeDtypeStruct((NP, 1), jnp.float32),
        ],
    )(cnt, x_pad, W1)


def _dense2_body(p_ref, g_ref, dinv_ref, b1_ref, w2_ref, b2_ref, o_ref):
    agg = (p_ref[0] + p_ref[1] + g_ref[...]) * dinv_ref[...]
    h1 = jnp.maximum(agg + b1_ref[...], 0.0)
    o = jnp.dot(h1, w2_ref[...], preferred_element_type=jnp.float32)
    o_ref[...] = jnp.maximum(o + b2_ref[...], 0.0)


def _dense2(p, g, dinv, b1, W2p, b2p):
    return pl.pallas_call(
        _dense2_body,
        grid=(NP // BR,),
        in_specs=[
            pl.BlockSpec((2, BR, HIDDEN), lambda i: (0, i, 0)),
            pl.BlockSpec((BR, HIDDEN), lambda i: (i, 0)),
            pl.BlockSpec((BR, 1), lambda i: (i, 0)),
            pl.BlockSpec((1, HIDDEN), lambda i: (0, 0)),
            pl.BlockSpec((HIDDEN, HIDDEN), lambda i: (0, 0)),
            pl.BlockSpec((1, HIDDEN), lambda i: (0, 0)),
        ],
        out_specs=pl.BlockSpec((BR, HIDDEN), lambda i: (i, 0)),
        out_shape=jax.ShapeDtypeStruct((NP, HIDDEN), jnp.float32),
    )(p, g, dinv, b1, W2p, b2p)


@jax.jit
def kernel(x, edge_index, W1, b1, W2, b2):
    src = edge_index[0]
    dst = edge_index[1]
    pad = jnp.full((E_PAD - E,), N, dtype=jnp.int32)
    src_pad = jnp.concatenate([src, pad])
    dst_pad = jnp.concatenate([dst, pad])
    dst3 = dst_pad.reshape(NW, NCH, CHUNK)
    packed3 = (src_pad | (dst_pad << 14)).reshape(NW, ENCH, ECHUNK)
    x_pad = jnp.pad(x, ((0, NP - N), (0, 0)))

    cnt = _deg_kernel(dst3).reshape(NC, NP, 1)
    g, dinv = _dense1(cnt, x_pad, W1)
    p = _edge_kernel(g, packed3).reshape(NC, NP, HIDDEN)

    b1r = b1.reshape(1, HIDDEN)
    W2p = jnp.pad(W2, ((0, 0), (0, HIDDEN - NCLASS)))
    b2p = jnp.pad(b2, (0, HIDDEN - NCLASS)).reshape(1, HIDDEN)
    out = _dense2(p, g, dinv, b1r, W2p, b2p)
    return out[:N, :NCLASS]


# speedup vs baseline: 1.0183x; 1.0183x over previous
import functools

import jax
import jax.numpy as jnp
from jax import lax
from jax.experimental import pallas as pl
from jax.experimental.pallas import tpu as pltpu
from jax.experimental.pallas import tpu_sc as plsc

N = 10000
E = 320000
NFEAT = 128
HIDDEN = 128
NCLASS = 16

NC = 2
NS = 16
NW = NC * NS
CHUNK = 128

NP = 10240
ROWS_PER_TILE = NP // NS
EPT = 10240
E_PAD = EPT * NW
NCH = EPT // CHUNK
DSEM = 8

ECHUNK = 64
NBUF = 4
LOOK = 2
ENCH0 = 160
ENCH1 = 160
TOTCH = NS * (ENCH0 + ENCH1)

_mesh = plsc.VectorSubcoreMesh(core_axis_name="c", subcore_axis_name="s")


def _fill_2d(ref, rows, value):
    vec = jnp.full((16,), value, dtype=jnp.float32)

    def body(i, _):
        r = i // 8
        col = (i % 8) * 16
        ref[r, pl.ds(col, 16)] = vec
        return 0

    lax.fori_loop(0, rows * 8, body, 0)


@functools.partial(
    pl.kernel,
    mesh=_mesh,
    out_type=jax.ShapeDtypeStruct((NC * NP,), jnp.float32),
    scratch_types=[
        pltpu.VMEM((CHUNK,), jnp.float32),
        pltpu.VMEM((NCH, CHUNK), jnp.int32),
        pltpu.VMEM((ROWS_PER_TILE,), jnp.float32),
        pltpu.VMEM_SHARED((NP,), jnp.float32),
    ] + [pltpu.SemaphoreType.DMA] * DSEM,
)
def _deg_kernel(dst_hbm, out_hbm, ones_v, idx_v, zero_v, acc_sh, *sems):
    c = lax.axis_index("c")
    s = lax.axis_index("s")
    wid = s * NC + c
    one = jnp.full((16,), 1.0, dtype=jnp.float32)
    zero = jnp.zeros((16,), dtype=jnp.float32)

    def fill_ones(i, _):
        ones_v[pl.ds(i * 16, 16)] = one
        return 0

    lax.fori_loop(0, CHUNK // 16, fill_ones, 0)

    def fill_zero(i, _):
        zero_v[pl.ds(i * 16, 16)] = zero
        return 0

    lax.fori_loop(0, ROWS_PER_TILE // 16, fill_zero, 0)
    pltpu.sync_copy(zero_v, acc_sh.at[pl.ds(s * ROWS_PER_TILE, ROWS_PER_TILE)])
    pltpu.sync_copy(dst_hbm.at[wid], idx_v)
    plsc.subcore_barrier()

    def grp_body(grp, _):
        for b in range(DSEM):
            i = grp * DSEM + b

            @pl.when(i >= DSEM)
            def _():
                pltpu.make_async_copy(ones_v, acc_sh.at[idx_v.at[0]],
                                      sems[b]).wait()

            pltpu.async_copy(ones_v, acc_sh.at[idx_v.at[i]], sems[b],
                             add=True)
        return 0

    lax.fori_loop(0, NCH // DSEM, grp_body, 0)
    for b in range(DSEM):
        pltpu.make_async_copy(ones_v, acc_sh.at[idx_v.at[0]], sems[b]).wait()
    plsc.subcore_barrier()

    row0 = s * ROWS_PER_TILE
    pltpu.sync_copy(
        acc_sh.at[pl.ds(row0, ROWS_PER_TILE)],
        out_hbm.at[pl.ds(c * NP + row0, ROWS_PER_TILE)],
    )


@functools.partial(
    pl.kernel,
    mesh=_mesh,
    out_type=jax.ShapeDtypeStruct((NC * NP, HIDDEN), jnp.float32),
    scratch_types=[pltpu.VMEM((ECHUNK, HIDDEN), jnp.float32)] * NBUF
      + [pltpu.VMEM((ECHUNK,), jnp.int32)] * NBUF
      + [pltpu.VMEM((ECHUNK,), jnp.int32)] * NBUF
      + [pltpu.VMEM((ECHUNK,), jnp.int32)] * NBUF
      + [pltpu.VMEM_SHARED((NP, HIDDEN), jnp.float32)]
      + [pltpu.SemaphoreType.DMA] * (3 * NBUF),
)
def _edge_kernel(g_hbm, pk_hbm, out_hbm, *rest):
    rows = list(rest[:NBUF])
    pk_r = list(rest[NBUF:2 * NBUF])
    idxs_r = list(rest[2 * NBUF:3 * NBUF])
    idxd_r = list(rest[3 * NBUF:4 * NBUF])
    acc_sh = rest[4 * NBUF]
    sems = rest[4 * NBUF + 1:]
    gsem = sems[:NBUF]
    ssem = sems[NBUF:2 * NBUF]
    isem = sems[2 * NBUF:]
    c = lax.axis_index("c")
    s = lax.axis_index("s")
    nch = jnp.where(c == 0, ENCH0, ENCH1)
    base_ch = jnp.where(c == 0, s * ENCH0, NS * ENCH0 + s * ENCH1)

    _fill_2d(rows[0], ECHUNK, 0.0)
    row0 = s * ROWS_PER_TILE
    NZ = ROWS_PER_TILE // ECHUNK

    def zbody(i, _):
        pltpu.async_copy(rows[0],
                         acc_sh.at[pl.ds(row0 + i * ECHUNK, ECHUNK), :],
                         gsem[0])
        return 0

    lax.fori_loop(0, NZ, zbody, 0)

    def zdrain(i, _):
        pltpu.make_async_copy(
            rows[0], acc_sh.at[pl.ds(row0, ECHUNK), :], gsem[0]).wait()
        return 0

    for b in range(NBUF):
        pltpu.async_copy(pk_hbm.at[base_ch + b], pk_r[b], isem[b])
    lax.fori_loop(0, NZ, zdrain, 0)
    plsc.subcore_barrier()

    def unpack(j, b):
        pltpu.make_async_copy(pk_hbm.at[base_ch], pk_r[b], isem[b]).wait()

        def ub(k, _):
            v = pk_r[b][pl.ds(k * 16, 16)]
            idxs_r[b][pl.ds(k * 16, 16)] = lax.bitwise_and(v, 0x3FFF)
            idxd_r[b][pl.ds(k * 16, 16)] = lax.shift_right_logical(v, 14)
            return 0

        lax.fori_loop(0, ECHUNK // 16, ub, 0)

        @pl.when(j + NBUF < nch)
        def _():
            pltpu.async_copy(pk_hbm.at[base_ch + j + NBUF], pk_r[b], isem[b])

    def scatter_wait(b):
        pltpu.make_async_copy(rows[b], acc_sh.at[idxd_r[b]], ssem[b]).wait()

    def gather_wait(b):
        pltpu.make_async_copy(g_hbm.at[idxs_r[b]], rows[b], gsem[b]).wait()

    for b in range(LOOK):
        unpack(b, b)
        pltpu.async_copy(g_hbm.at[idxs_r[b]], rows[b], gsem[b])

    def grp_body(grp, _):
        for b in range(NBUF):
            i = grp * NBUF + b
            bl = (b + LOOK) % NBUF

            @pl.when(i + LOOK < nch)
            def _():
                @pl.when(i + LOOK >= NBUF)
                def _():
                    scatter_wait(bl)

                unpack(i + LOOK, bl)
                pltpu.async_copy(g_hbm.at[idxs_r[bl]], rows[bl], gsem[bl])

            gather_wait(b)
            pltpu.async_copy(rows[b], acc_sh.at[idxd_r[b]], ssem[b],
                             add=True)
        return 0

    lax.fori_loop(0, nch // NBUF, grp_body, 0)
    for b in range(NBUF):
        scatter_wait(b)
    plsc.subcore_barrier()

    def obody(i, _):
        r = row0 + i * ECHUNK
        pltpu.async_copy(acc_sh.at[pl.ds(r, ECHUNK), :],
                         out_hbm.at[pl.ds(c * NP + r, ECHUNK), :], gsem[0])
        return 0

    lax.fori_loop(0, NZ, obody, 0)

    def odrain(i, _):
        pltpu.make_async_copy(
            acc_sh.at[pl.ds(row0, ECHUNK), :],
            out_hbm.at[pl.ds(c * NP, ECHUNK), :], gsem[0]).wait()
        return 0

    lax.fori_loop(0, NZ, odrain, 0)


BR = 640


def _dense1_body(cnt_ref, x_ref, w1_ref, g_ref, dinv_ref):
    deg = cnt_ref[0] + cnt_ref[1] + 1.0
    dinv = lax.rsqrt(deg)
    h = jnp.dot(x_ref[...], w1_ref[...], preferred_element_type=jnp.float32)
    g_ref[...] = h * dinv
    dinv_ref[...] = dinv


def _dense1(cnt, x_pad, W1):
    return pl.pallas_call(
        _dense1_body,
        grid=(NP // BR,),
        in_specs=[
            pl.BlockSpec((2, BR, 1), lambda i: (0, i, 0)),
            pl.BlockSpec((BR, NFEAT), lambda i: (i, 0)),
            pl.BlockSpec((NFEAT, HIDDEN), lambda i: (0, 0)),
        ],
        out_specs=[
            pl.BlockSpec((BR, HIDDEN), lambda i: (i, 0)),
            pl.BlockSpec((BR, 1), lambda i: (i, 0)),
        ],
        out_shape=[
            jax.ShapeDtypeStruct((NP, HIDDEN), jnp.float32),
            jax.ShapeDtypeStruct((NP, 1), jnp.float32),
        ],
    )(cnt, x_pad, W1)


def _dense2_body(p_ref, g_ref, dinv_ref, b1_ref, w2_ref, b2_ref, o_ref):
    agg = (p_ref[0] + p_ref[1] + g_ref[...]) * dinv_ref[...]
    h1 = jnp.maximum(agg + b1_ref[...], 0.0)
    o = jnp.dot(h1, w2_ref[...], preferred_element_type=jnp.float32)
    o_ref[...] = jnp.maximum(o + b2_ref[...], 0.0)


def _dense2(p, g, dinv, b1, W2p, b2p):
    return pl.pallas_call(
        _dense2_body,
        grid=(NP // BR,),
        in_specs=[
            pl.BlockSpec((2, BR, HIDDEN), lambda i: (0, i, 0)),
            pl.BlockSpec((BR, HIDDEN), lambda i: (i, 0)),
            pl.BlockSpec((BR, 1), lambda i: (i, 0)),
            pl.BlockSpec((1, HIDDEN), lambda i: (0, 0)),
            pl.BlockSpec((HIDDEN, HIDDEN), lambda i: (0, 0)),
            pl.BlockSpec((1, HIDDEN), lambda i: (0, 0)),
        ],
        out_specs=pl.BlockSpec((BR, HIDDEN), lambda i: (i, 0)),
        out_shape=jax.ShapeDtypeStruct((NP, HIDDEN), jnp.float32),
    )(p, g, dinv, b1, W2p, b2p)


@jax.jit
def kernel(x, edge_index, W1, b1, W2, b2):
    src = edge_index[0]
    dst = edge_index[1]
    npad = E_PAD - E
    spread = jnp.arange(npad, dtype=jnp.int32)
    src_pad = jnp.concatenate([src, jnp.full((npad,), N, dtype=jnp.int32)])
    dst_deg = jnp.concatenate([dst, N + spread % (NP - N)])
    dst_edge = jnp.concatenate([dst, spread % N])
    dst3 = dst_deg.reshape(NW, NCH, CHUNK)
    packed2 = (src_pad | (dst_edge << 14)).reshape(TOTCH, ECHUNK)
    x_pad = jnp.pad(x, ((0, NP - N), (0, 0)))

    cnt = _deg_kernel(dst3).reshape(NC, NP, 1)
    g, dinv = _dense1(cnt, x_pad, W1)
    p = _edge_kernel(g, packed2).reshape(NC, NP, HIDDEN)

    b1r = b1.reshape(1, HIDDEN)
    W2p = jnp.pad(W2, ((0, 0), (0, HIDDEN - NCLASS)))
    b2p = jnp.pad(b2, (0, HIDDEN - NCLASS)).reshape(1, HIDDEN)
    out = _dense2(p, g, dinv, b1r, W2p, b2p)
    return out[:N, :NCLASS]


# speedup vs baseline: 1.0670x; 1.0478x over previous
import functools

import jax
import jax.numpy as jnp
from jax import lax
from jax.experimental import pallas as pl
from jax.experimental.pallas import tpu as pltpu
from jax.experimental.pallas import tpu_sc as plsc

N = 10000
E = 320000
NFEAT = 128
HIDDEN = 128
NCLASS = 16

NC = 2
NS = 16
NW = NC * NS
CHUNK = 128

NP = 10240
ROWS_PER_TILE = NP // NS
EPT = 10240
E_PAD = EPT * NW
NCH = EPT // CHUNK
DSEM = 8

ECHUNK = 64
NBUF = 4
LOOK = 2
ENCH0 = 260
ENCH1 = 60
TOTCH = NS * (ENCH0 + ENCH1)

_mesh = plsc.VectorSubcoreMesh(core_axis_name="c", subcore_axis_name="s")


def _fill_2d(ref, rows, value):
    vec = jnp.full((16,), value, dtype=jnp.float32)

    def body(i, _):
        r = i // 8
        col = (i % 8) * 16
        ref[r, pl.ds(col, 16)] = vec
        return 0

    lax.fori_loop(0, rows * 8, body, 0)


@functools.partial(
    pl.kernel,
    mesh=_mesh,
    out_type=jax.ShapeDtypeStruct((NC * NP,), jnp.float32),
    scratch_types=[
        pltpu.VMEM((CHUNK,), jnp.float32),
        pltpu.VMEM((NCH, CHUNK), jnp.int32),
        pltpu.VMEM((ROWS_PER_TILE,), jnp.float32),
        pltpu.VMEM_SHARED((NP,), jnp.float32),
    ] + [pltpu.SemaphoreType.DMA] * DSEM,
)
def _deg_kernel(dst_hbm, out_hbm, ones_v, idx_v, zero_v, acc_sh, *sems):
    c = lax.axis_index("c")
    s = lax.axis_index("s")
    wid = s * NC + c
    one = jnp.full((16,), 1.0, dtype=jnp.float32)
    zero = jnp.zeros((16,), dtype=jnp.float32)

    def fill_ones(i, _):
        ones_v[pl.ds(i * 16, 16)] = one
        return 0

    lax.fori_loop(0, CHUNK // 16, fill_ones, 0)

    def fill_zero(i, _):
        zero_v[pl.ds(i * 16, 16)] = zero
        return 0

    lax.fori_loop(0, ROWS_PER_TILE // 16, fill_zero, 0)
    pltpu.sync_copy(zero_v, acc_sh.at[pl.ds(s * ROWS_PER_TILE, ROWS_PER_TILE)])
    pltpu.sync_copy(dst_hbm.at[wid], idx_v)
    plsc.subcore_barrier()

    def grp_body(grp, _):
        for b in range(DSEM):
            i = grp * DSEM + b

            @pl.when(i >= DSEM)
            def _():
                pltpu.make_async_copy(ones_v, acc_sh.at[idx_v.at[0]],
                                      sems[b]).wait()

            pltpu.async_copy(ones_v, acc_sh.at[idx_v.at[i]], sems[b],
                             add=True)
        return 0

    lax.fori_loop(0, NCH // DSEM, grp_body, 0)
    for b in range(DSEM):
        pltpu.make_async_copy(ones_v, acc_sh.at[idx_v.at[0]], sems[b]).wait()
    plsc.subcore_barrier()

    row0 = s * ROWS_PER_TILE
    pltpu.sync_copy(
        acc_sh.at[pl.ds(row0, ROWS_PER_TILE)],
        out_hbm.at[pl.ds(c * NP + row0, ROWS_PER_TILE)],
    )


@functools.partial(
    pl.kernel,
    mesh=_mesh,
    out_type=jax.ShapeDtypeStruct((NC * NP, HIDDEN), jnp.float32),
    scratch_types=[pltpu.VMEM((ECHUNK, HIDDEN), jnp.float32)] * NBUF
      + [pltpu.VMEM((ECHUNK,), jnp.int32)] * NBUF
      + [pltpu.VMEM((ECHUNK,), jnp.int32)] * NBUF
      + [pltpu.VMEM((ECHUNK,), jnp.int32)] * NBUF
      + [pltpu.VMEM_SHARED((NP, HIDDEN), jnp.float32)]
      + [pltpu.SemaphoreType.DMA] * (3 * NBUF),
)
def _edge_kernel(g_hbm, pk_hbm, out_hbm, *rest):
    rows = list(rest[:NBUF])
    pk_r = list(rest[NBUF:2 * NBUF])
    idxs_r = list(rest[2 * NBUF:3 * NBUF])
    idxd_r = list(rest[3 * NBUF:4 * NBUF])
    acc_sh = rest[4 * NBUF]
    sems = rest[4 * NBUF + 1:]
    gsem = sems[:NBUF]
    ssem = sems[NBUF:2 * NBUF]
    isem = sems[2 * NBUF:]
    c = lax.axis_index("c")
    s = lax.axis_index("s")
    nch = jnp.where(c == 0, ENCH0, ENCH1)
    base_ch = jnp.where(c == 0, s * ENCH0, NS * ENCH0 + s * ENCH1)

    _fill_2d(rows[0], ECHUNK, 0.0)
    row0 = s * ROWS_PER_TILE
    NZ = ROWS_PER_TILE // ECHUNK

    def zbody(i, _):
        pltpu.async_copy(rows[0],
                         acc_sh.at[pl.ds(row0 + i * ECHUNK, ECHUNK), :],
                         gsem[0])
        return 0

    lax.fori_loop(0, NZ, zbody, 0)

    def zdrain(i, _):
        pltpu.make_async_copy(
            rows[0], acc_sh.at[pl.ds(row0, ECHUNK), :], gsem[0]).wait()
        return 0

    for b in range(NBUF):
        pltpu.async_copy(pk_hbm.at[base_ch + b], pk_r[b], isem[b])
    lax.fori_loop(0, NZ, zdrain, 0)
    plsc.subcore_barrier()

    def unpack(j, b):
        pltpu.make_async_copy(pk_hbm.at[base_ch], pk_r[b], isem[b]).wait()

        def ub(k, _):
            v = pk_r[b][pl.ds(k * 16, 16)]
            idxs_r[b][pl.ds(k * 16, 16)] = lax.bitwise_and(v, 0x3FFF)
            idxd_r[b][pl.ds(k * 16, 16)] = lax.shift_right_logical(v, 14)
            return 0

        lax.fori_loop(0, ECHUNK // 16, ub, 0)

        @pl.when(j + NBUF < nch)
        def _():
            pltpu.async_copy(pk_hbm.at[base_ch + j + NBUF], pk_r[b], isem[b])

    def scatter_wait(b):
        pltpu.make_async_copy(rows[b], acc_sh.at[idxd_r[b]], ssem[b]).wait()

    def gather_wait(b):
        pltpu.make_async_copy(g_hbm.at[idxs_r[b]], rows[b], gsem[b]).wait()

    for b in range(LOOK):
        unpack(b, b)
        pltpu.async_copy(g_hbm.at[idxs_r[b]], rows[b], gsem[b])

    def grp_body(grp, _):
        for b in range(NBUF):
            i = grp * NBUF + b
            bl = (b + LOOK) % NBUF

            @pl.when(i + LOOK < nch)
            def _():
                @pl.when(i + LOOK >= NBUF)
                def _():
                    scatter_wait(bl)

                unpack(i + LOOK, bl)
                pltpu.async_copy(g_hbm.at[idxs_r[bl]], rows[bl], gsem[bl])

            gather_wait(b)
            pltpu.async_copy(rows[b], acc_sh.at[idxd_r[b]], ssem[b],
                             add=True)
        return 0

    lax.fori_loop(0, nch // NBUF, grp_body, 0)
    for b in range(NBUF):
        scatter_wait(b)
    plsc.subcore_barrier()

    def obody(i, _):
        r = row0 + i * ECHUNK
        pltpu.async_copy(acc_sh.at[pl.ds(r, ECHUNK), :],
                         out_hbm.at[pl.ds(c * NP + r, ECHUNK), :], gsem[0])
        return 0

    lax.fori_loop(0, NZ, obody, 0)

    def odrain(i, _):
        pltpu.make_async_copy(
            acc_sh.at[pl.ds(row0, ECHUNK), :],
            out_hbm.at[pl.ds(c * NP, ECHUNK), :], gsem[0]).wait()
        return 0

    lax.fori_loop(0, NZ, odrain, 0)


BR = 640


def _dense1_body(cnt_ref, x_ref, w1_ref, g_ref, dinv_ref):
    deg = cnt_ref[0] + cnt_ref[1] + 1.0
    dinv = lax.rsqrt(deg)
    h = jnp.dot(x_ref[...], w1_ref[...], preferred_element_type=jnp.float32)
    g_ref[...] = h * dinv
    dinv_ref[...] = dinv


def _dense1(cnt, x_pad, W1):
    return pl.pallas_call(
        _dense1_body,
        grid=(NP // BR,),
        in_specs=[
            pl.BlockSpec((2, BR, 1), lambda i: (0, i, 0)),
            pl.BlockSpec((BR, NFEAT), lambda i: (i, 0)),
            pl.BlockSpec((NFEAT, HIDDEN), lambda i: (0, 0)),
        ],
        out_specs=[
            pl.BlockSpec((BR, HIDDEN), lambda i: (i, 0)),
            pl.BlockSpec((BR, 1), lambda i: (i, 0)),
        ],
        out_shape=[
            jax.ShapeDtypeStruct((NP, HIDDEN), jnp.float32),
            jax.ShapeDtypeStruct((NP, 1), jnp.float32),
        ],
    )(cnt, x_pad, W1)


def _dense2_body(p_ref, g_ref, dinv_ref, b1_ref, w2_ref, b2_ref, o_ref):
    agg = (p_ref[0] + p_ref[1] + g_ref[...]) * dinv_ref[...]
    h1 = jnp.maximum(agg + b1_ref[...], 0.0)
    o = jnp.dot(h1, w2_ref[...], preferred_element_type=jnp.float32)
    o_ref[...] = jnp.maximum(o + b2_ref[...], 0.0)


def _dense2(p, g, dinv, b1, W2p, b2p):
    return pl.pallas_call(
        _dense2_body,
        grid=(NP // BR,),
        in_specs=[
            pl.BlockSpec((2, BR, HIDDEN), lambda i: (0, i, 0)),
            pl.BlockSpec((BR, HIDDEN), lambda i: (i, 0)),
            pl.BlockSpec((BR, 1), lambda i: (i, 0)),
            pl.BlockSpec((1, HIDDEN), lambda i: (0, 0)),
            pl.BlockSpec((HIDDEN, HIDDEN), lambda i: (0, 0)),
            pl.BlockSpec((1, HIDDEN), lambda i: (0, 0)),
        ],
        out_specs=pl.BlockSpec((BR, HIDDEN), lambda i: (i, 0)),
        out_shape=jax.ShapeDtypeStruct((NP, HIDDEN), jnp.float32),
    )(p, g, dinv, b1, W2p, b2p)


@jax.jit
def kernel(x, edge_index, W1, b1, W2, b2):
    src = edge_index[0]
    dst = edge_index[1]
    npad = E_PAD - E
    spread = jnp.arange(npad, dtype=jnp.int32)
    src_pad = jnp.concatenate([src, jnp.full((npad,), N, dtype=jnp.int32)])
    dst_deg = jnp.concatenate([dst, N + spread % (NP - N)])
    dst_edge = jnp.concatenate([dst, spread % N])
    dst3 = dst_deg.reshape(NW, NCH, CHUNK)
    packed2 = (src_pad | (dst_edge << 14)).reshape(TOTCH, ECHUNK)
    x_pad = jnp.pad(x, ((0, NP - N), (0, 0)))

    cnt = _deg_kernel(dst3).reshape(NC, NP, 1)
    g, dinv = _dense1(cnt, x_pad, W1)
    p = _edge_kernel(g, packed2).reshape(NC, NP, HIDDEN)

    b1r = b1.reshape(1, HIDDEN)
    W2p = jnp.pad(W2, ((0, 0), (0, HIDDEN - NCLASS)))
    b2p = jnp.pad(b2, (0, HIDDEN - NCLASS)).reshape(1, HIDDEN)
    out = _dense2(p, g, dinv, b1r, W2p, b2p)
    return out[:N, :NCLASS]


# speedup vs baseline: 1.0922x; 1.0236x over previous
import functools

import jax
import jax.numpy as jnp
from jax import lax
from jax.experimental import pallas as pl
from jax.experimental.pallas import tpu as pltpu
from jax.experimental.pallas import tpu_sc as plsc

N = 10000
E = 320000
NFEAT = 128
HIDDEN = 128
NCLASS = 16

NC = 2
NS = 16
NW = NC * NS
CHUNK = 128

NP = 10240
ROWS_PER_TILE = NP // NS
EPT = 10240
E_PAD = EPT * NW
NCH = EPT // CHUNK
DSEM = 8

ECHUNK = 64
NBUF = 4
LOOK = 2
ENCH0 = 316
ENCH1 = 4
TOTCH = NS * (ENCH0 + ENCH1)

_mesh = plsc.VectorSubcoreMesh(core_axis_name="c", subcore_axis_name="s")


def _fill_2d(ref, rows, value):
    vec = jnp.full((16,), value, dtype=jnp.float32)

    def body(i, _):
        r = i // 8
        col = (i % 8) * 16
        ref[r, pl.ds(col, 16)] = vec
        return 0

    lax.fori_loop(0, rows * 8, body, 0)


@functools.partial(
    pl.kernel,
    mesh=_mesh,
    out_type=jax.ShapeDtypeStruct((NC * NP,), jnp.float32),
    scratch_types=[
        pltpu.VMEM((CHUNK,), jnp.float32),
        pltpu.VMEM((NCH, CHUNK), jnp.int32),
        pltpu.VMEM((ROWS_PER_TILE,), jnp.float32),
        pltpu.VMEM_SHARED((NP,), jnp.float32),
    ] + [pltpu.SemaphoreType.DMA] * DSEM,
)
def _deg_kernel(dst_hbm, out_hbm, ones_v, idx_v, zero_v, acc_sh, *sems):
    c = lax.axis_index("c")
    s = lax.axis_index("s")
    wid = s * NC + c
    one = jnp.full((16,), 1.0, dtype=jnp.float32)
    zero = jnp.zeros((16,), dtype=jnp.float32)

    def fill_ones(i, _):
        ones_v[pl.ds(i * 16, 16)] = one
        return 0

    lax.fori_loop(0, CHUNK // 16, fill_ones, 0)

    def fill_zero(i, _):
        zero_v[pl.ds(i * 16, 16)] = zero
        return 0

    lax.fori_loop(0, ROWS_PER_TILE // 16, fill_zero, 0)
    pltpu.sync_copy(zero_v, acc_sh.at[pl.ds(s * ROWS_PER_TILE, ROWS_PER_TILE)])
    pltpu.sync_copy(dst_hbm.at[wid], idx_v)
    plsc.subcore_barrier()

    def grp_body(grp, _):
        for b in range(DSEM):
            i = grp * DSEM + b

            @pl.when(i >= DSEM)
            def _():
                pltpu.make_async_copy(ones_v, acc_sh.at[idx_v.at[0]],
                                      sems[b]).wait()

            pltpu.async_copy(ones_v, acc_sh.at[idx_v.at[i]], sems[b],
                             add=True)
        return 0

    lax.fori_loop(0, NCH // DSEM, grp_body, 0)
    for b in range(DSEM):
        pltpu.make_async_copy(ones_v, acc_sh.at[idx_v.at[0]], sems[b]).wait()
    plsc.subcore_barrier()

    row0 = s * ROWS_PER_TILE
    pltpu.sync_copy(
        acc_sh.at[pl.ds(row0, ROWS_PER_TILE)],
        out_hbm.at[pl.ds(c * NP + row0, ROWS_PER_TILE)],
    )


@functools.partial(
    pl.kernel,
    mesh=_mesh,
    out_type=jax.ShapeDtypeStruct((NC * NP, HIDDEN), jnp.float32),
    scratch_types=[pltpu.VMEM((ECHUNK, HIDDEN), jnp.float32)] * NBUF
      + [pltpu.VMEM((ECHUNK,), jnp.int32)] * NBUF
      + [pltpu.VMEM((ECHUNK,), jnp.int32)] * NBUF
      + [pltpu.VMEM((ECHUNK,), jnp.int32)] * NBUF
      + [pltpu.VMEM_SHARED((NP, HIDDEN), jnp.float32)]
      + [pltpu.SemaphoreType.DMA] * (3 * NBUF),
)
def _edge_kernel(g_hbm, pk_hbm, out_hbm, *rest):
    rows = list(rest[:NBUF])
    pk_r = list(rest[NBUF:2 * NBUF])
    idxs_r = list(rest[2 * NBUF:3 * NBUF])
    idxd_r = list(rest[3 * NBUF:4 * NBUF])
    acc_sh = rest[4 * NBUF]
    sems = rest[4 * NBUF + 1:]
    gsem = sems[:NBUF]
    ssem = sems[NBUF:2 * NBUF]
    isem = sems[2 * NBUF:]
    c = lax.axis_index("c")
    s = lax.axis_index("s")
    nch = jnp.where(c == 0, ENCH0, ENCH1)
    base_ch = jnp.where(c == 0, s * ENCH0, NS * ENCH0 + s * ENCH1)

    _fill_2d(rows[0], ECHUNK, 0.0)
    row0 = s * ROWS_PER_TILE
    NZ = ROWS_PER_TILE // ECHUNK

    def zbody(i, _):
        pltpu.async_copy(rows[0],
                         acc_sh.at[pl.ds(row0 + i * ECHUNK, ECHUNK), :],
                         gsem[0])
        return 0

    lax.fori_loop(0, NZ, zbody, 0)

    def zdrain(i, _):
        pltpu.make_async_copy(
            rows[0], acc_sh.at[pl.ds(row0, ECHUNK), :], gsem[0]).wait()
        return 0

    for b in range(NBUF):
        pltpu.async_copy(pk_hbm.at[base_ch + b], pk_r[b], isem[b])
    lax.fori_loop(0, NZ, zdrain, 0)
    plsc.subcore_barrier()

    def unpack(j, b):
        pltpu.make_async_copy(pk_hbm.at[base_ch], pk_r[b], isem[b]).wait()

        def ub(k, _):
            v = pk_r[b][pl.ds(k * 16, 16)]
            idxs_r[b][pl.ds(k * 16, 16)] = lax.bitwise_and(v, 0x3FFF)
            idxd_r[b][pl.ds(k * 16, 16)] = lax.shift_right_logical(v, 14)
            return 0

        lax.fori_loop(0, ECHUNK // 16, ub, 0)

        @pl.when(j + NBUF < nch)
        def _():
            pltpu.async_copy(pk_hbm.at[base_ch + j + NBUF], pk_r[b], isem[b])

    def scatter_wait(b):
        pltpu.make_async_copy(rows[b], acc_sh.at[idxd_r[b]], ssem[b]).wait()

    def gather_wait(b):
        pltpu.make_async_copy(g_hbm.at[idxs_r[b]], rows[b], gsem[b]).wait()

    for b in range(LOOK):
        unpack(b, b)
        pltpu.async_copy(g_hbm.at[idxs_r[b]], rows[b], gsem[b])

    def grp_body(grp, _):
        for b in range(NBUF):
            i = grp * NBUF + b
            bl = (b + LOOK) % NBUF

            @pl.when(i + LOOK < nch)
            def _():
                @pl.when(i + LOOK >= NBUF)
                def _():
                    scatter_wait(bl)

                unpack(i + LOOK, bl)
                pltpu.async_copy(g_hbm.at[idxs_r[bl]], rows[bl], gsem[bl])

            gather_wait(b)
            pltpu.async_copy(rows[b], acc_sh.at[idxd_r[b]], ssem[b],
                             add=True)
        return 0

    lax.fori_loop(0, nch // NBUF, grp_body, 0)
    for b in range(NBUF):
        scatter_wait(b)
    plsc.subcore_barrier()

    def obody(i, _):
        r = row0 + i * ECHUNK
        pltpu.async_copy(acc_sh.at[pl.ds(r, ECHUNK), :],
                         out_hbm.at[pl.ds(c * NP + r, ECHUNK), :], gsem[0])
        return 0

    lax.fori_loop(0, NZ, obody, 0)

    def odrain(i, _):
        pltpu.make_async_copy(
            acc_sh.at[pl.ds(row0, ECHUNK), :],
            out_hbm.at[pl.ds(c * NP, ECHUNK), :], gsem[0]).wait()
        return 0

    lax.fori_loop(0, NZ, odrain, 0)


BR = 640


def _dense1_body(cnt_ref, x_ref, w1_ref, g_ref, dinv_ref):
    deg = cnt_ref[0] + cnt_ref[1] + 1.0
    dinv = lax.rsqrt(deg)
    h = jnp.dot(x_ref[...], w1_ref[...], preferred_element_type=jnp.float32)
    g_ref[...] = h * dinv
    dinv_ref[...] = dinv


def _dense1(cnt, x_pad, W1):
    return pl.pallas_call(
        _dense1_body,
        grid=(NP // BR,),
        in_specs=[
            pl.BlockSpec((2, BR, 1), lambda i: (0, i, 0)),
            pl.BlockSpec((BR, NFEAT), lambda i: (i, 0)),
            pl.BlockSpec((NFEAT, HIDDEN), lambda i: (0, 0)),
        ],
        out_specs=[
            pl.BlockSpec((BR, HIDDEN), lambda i: (i, 0)),
            pl.BlockSpec((BR, 1), lambda i: (i, 0)),
        ],
        out_shape=[
            jax.ShapeDtypeStruct((NP, HIDDEN), jnp.float32),
            jax.ShapeDtypeStruct((NP, 1), jnp.float32),
        ],
    )(cnt, x_pad, W1)


def _dense2_body(p_ref, g_ref, dinv_ref, b1_ref, w2_ref, b2_ref, o_ref):
    agg = (p_ref[0] + p_ref[1] + g_ref[...]) * dinv_ref[...]
    h1 = jnp.maximum(agg + b1_ref[...], 0.0)
    o = jnp.dot(h1, w2_ref[...], preferred_element_type=jnp.float32)
    o_ref[...] = jnp.maximum(o + b2_ref[...], 0.0)


def _dense2(p, g, dinv, b1, W2p, b2p):
    return pl.pallas_call(
        _dense2_body,
        grid=(NP // BR,),
        in_specs=[
            pl.BlockSpec((2, BR, HIDDEN), lambda i: (0, i, 0)),
            pl.BlockSpec((BR, HIDDEN), lambda i: (i, 0)),
            pl.BlockSpec((BR, 1), lambda i: (i, 0)),
            pl.BlockSpec((1, HIDDEN), lambda i: (0, 0)),
            pl.BlockSpec((HIDDEN, HIDDEN), lambda i: (0, 0)),
            pl.BlockSpec((1, HIDDEN), lambda i: (0, 0)),
        ],
        out_specs=pl.BlockSpec((BR, HIDDEN), lambda i: (i, 0)),
        out_shape=jax.ShapeDtypeStruct((NP, HIDDEN), jnp.float32),
    )(p, g, dinv, b1, W2p, b2p)


@jax.jit
def kernel(x, edge_index, W1, b1, W2, b2):
    src = edge_index[0]
    dst = edge_index[1]
    npad = E_PAD - E
    spread = jnp.arange(npad, dtype=jnp.int32)
    src_pad = jnp.concatenate([src, jnp.full((npad,), N, dtype=jnp.int32)])
    dst_deg = jnp.concatenate([dst, N + spread % (NP - N)])
    dst_edge = jnp.concatenate([dst, spread % N])
    dst3 = dst_deg.reshape(NW, NCH, CHUNK)
    packed2 = (src_pad | (dst_edge << 14)).reshape(TOTCH, ECHUNK)
    x_pad = jnp.pad(x, ((0, NP - N), (0, 0)))

    cnt = _deg_kernel(dst3).reshape(NC, NP, 1)
    g, dinv = _dense1(cnt, x_pad, W1)
    p = _edge_kernel(g, packed2).reshape(NC, NP, HIDDEN)

    b1r = b1.reshape(1, HIDDEN)
    W2p = jnp.pad(W2, ((0, 0), (0, HIDDEN - NCLASS)))
    b2p = jnp.pad(b2, (0, HIDDEN - NCLASS)).reshape(1, HIDDEN)
    out = _dense2(p, g, dinv, b1r, W2p, b2p)
    return out[:N, :NCLASS]
